# baseline (speedup 1.0000x reference)
import jax
import jax.numpy as jnp
from jax.experimental import pallas as pl
from jax.experimental.pallas import tpu as pltpu

_N_CHUNKS = 4


def _mlp_body(x_ref, w1_ref, b1_ref, w2_ref, b2_ref, o_ref):
    h = jnp.dot(x_ref[0], w1_ref[:], preferred_element_type=jnp.float32)
    h = h + b1_ref[:]
    h = jnp.where(h > 0, h, jnp.exp(h) - 1.0)
    o = jnp.dot(h, w2_ref[:], preferred_element_type=jnp.float32)
    o_ref[0] = o + b2_ref[:]


def kernel(x, edge_index, W1, b1, W2, b2):
    del edge_index
    N, D = x.shape
    rows = N // _N_CHUNKS
    x3 = x.reshape(_N_CHUNKS, rows, D)
    b1_2d = b1.reshape(1, D)
    b2_2d = b2.reshape(1, D)
    out3 = pl.pallas_call(
        _mlp_body,
        grid=(_N_CHUNKS,),
        in_specs=[
            pl.BlockSpec((1, rows, D), lambda i: (i, 0, 0)),
            pl.BlockSpec((D, D), lambda i: (0, 0)),
            pl.BlockSpec((1, D), lambda i: (0, 0)),
            pl.BlockSpec((D, D), lambda i: (0, 0)),
            pl.BlockSpec((1, D), lambda i: (0, 0)),
        ],
        out_specs=pl.BlockSpec((1, rows, D), lambda i: (i, 0, 0)),
        out_shape=jax.ShapeDtypeStruct((_N_CHUNKS, rows, D), x.dtype),
        compiler_params=pltpu.CompilerParams(
            dimension_semantics=("arbitrary",)),
    )(x3, W1, b1_2d, W2, b2_2d)
    return out3.reshape(N, D)


# speedup vs baseline: 1.7322x; 1.7322x over previous
import jax
import jax.numpy as jnp
from jax.experimental import pallas as pl
from jax.experimental.pallas import tpu as pltpu

_BLOCK_ROWS = 2000


def _mlp_body(x_ref, w1_hbm, b1_hbm, w2_hbm, b2_hbm, o_ref,
              w1_v, b1_v, w2_v, b2_v, sem):
    @pl.when(pl.program_id(0) == 0)
    def _load_weights():
        c0 = pltpu.make_async_copy(w1_hbm, w1_v, sem.at[0])
        c1 = pltpu.make_async_copy(b1_hbm, b1_v, sem.at[1])
        c2 = pltpu.make_async_copy(w2_hbm, w2_v, sem.at[2])
        c3 = pltpu.make_async_copy(b2_hbm, b2_v, sem.at[3])
        c0.start()
        c1.start()
        c2.start()
        c3.start()
        c0.wait()
        c1.wait()
        c2.wait()
        c3.wait()

    h = jnp.dot(x_ref[:].astype(jnp.bfloat16),
                w1_v[:].astype(jnp.bfloat16),
                preferred_element_type=jnp.float32)
    h = h + b1_v[:]
    h = jnp.where(h > 0, h, jnp.exp(h) - 1.0)
    o = jnp.dot(h.astype(jnp.bfloat16),
                w2_v[:].astype(jnp.bfloat16),
                preferred_element_type=jnp.float32)
    o_ref[:] = o + b2_v[:]


def kernel(x, edge_index, W1, b1, W2, b2):
    del edge_index
    N, D = x.shape
    b1_2d = b1.reshape(1, D)
    b2_2d = b2.reshape(1, D)
    return pl.pallas_call(
        _mlp_body,
        grid=(N // _BLOCK_ROWS,),
        in_specs=[
            pl.BlockSpec((_BLOCK_ROWS, D), lambda i: (i, 0)),
            pl.BlockSpec(memory_space=pltpu.MemorySpace.HBM),
            pl.BlockSpec(memory_space=pltpu.MemorySpace.HBM),
            pl.BlockSpec(memory_space=pltpu.MemorySpace.HBM),
            pl.BlockSpec(memory_space=pltpu.MemorySpace.HBM),
        ],
        out_specs=pl.BlockSpec((_BLOCK_ROWS, D), lambda i: (i, 0)),
        out_shape=jax.ShapeDtypeStruct((N, D), x.dtype),
        scratch_shapes=[
            pltpu.VMEM((D, D), jnp.float32),
            pltpu.VMEM((1, D), jnp.float32),
            pltpu.VMEM((D, D), jnp.float32),
            pltpu.VMEM((1, D), jnp.float32),
            pltpu.SemaphoreType.DMA((4,)),
        ],
        compiler_params=pltpu.CompilerParams(
            dimension_semantics=("arbitrary",)),
    )(x, W1, b1_2d, W2, b2_2d)


# speedup vs baseline: 1.8233x; 1.0526x over previous
import jax
import jax.numpy as jnp
from jax.experimental import pallas as pl
from jax.experimental.pallas import tpu as pltpu

_CHUNK = 1000
_NBUF = 4
_LOOKAHEAD = 3
_D = 128


def _mlp_body(x_hbm, w1_ref, b1_ref, w2_ref, b2_ref, o_hbm,
              ibuf, obuf, in_sem, out_sem):
    n_chunks = x_hbm.shape[0] // _CHUNK

    def in_copy(k):
        s = k % _NBUF
        return pltpu.make_async_copy(
            x_hbm.at[pl.ds(k * _CHUNK, _CHUNK), :], ibuf.at[s], in_sem.at[s])

    def out_copy(k):
        s = k % _NBUF
        return pltpu.make_async_copy(
            obuf.at[s], o_hbm.at[pl.ds(k * _CHUNK, _CHUNK), :], out_sem.at[s])

    for k in range(min(_LOOKAHEAD, n_chunks)):
        in_copy(k).start()
    for k in range(n_chunks):
        s = k % _NBUF
        if k + _LOOKAHEAD < n_chunks:
            in_copy(k + _LOOKAHEAD).start()
        in_copy(k).wait()
        if k >= _NBUF:
            out_copy(k - _NBUF).wait()
        h = jnp.dot(ibuf[s].astype(jnp.bfloat16),
                    w1_ref[:].astype(jnp.bfloat16),
                    preferred_element_type=jnp.float32)
        h = h + b1_ref[:]
        h = jnp.where(h > 0, h, jnp.exp(h) - 1.0)
        o = jnp.dot(h.astype(jnp.bfloat16),
                    w2_ref[:].astype(jnp.bfloat16),
                    preferred_element_type=jnp.float32)
        obuf[s] = o + b2_ref[:]
        out_copy(k).start()
    for k in range(max(n_chunks - _NBUF, 0), n_chunks):
        out_copy(k).wait()


def kernel(x, edge_index, W1, b1, W2, b2):
    del edge_index
    N, D = x.shape
    b1_2d = b1.reshape(1, D)
    b2_2d = b2.reshape(1, D)
    return pl.pallas_call(
        _mlp_body,
        in_specs=[
            pl.BlockSpec(memory_space=pltpu.MemorySpace.HBM),
            pl.BlockSpec(memory_space=pltpu.MemorySpace.VMEM),
            pl.BlockSpec(memory_space=pltpu.MemorySpace.VMEM),
            pl.BlockSpec(memory_space=pltpu.MemorySpace.VMEM),
            pl.BlockSpec(memory_space=pltpu.MemorySpace.VMEM),
        ],
        out_specs=pl.BlockSpec(memory_space=pltpu.MemorySpace.HBM),
        out_shape=jax.ShapeDtypeStruct((N, D), x.dtype),
        scratch_shapes=[
            pltpu.VMEM((_NBUF, _CHUNK, _D), jnp.float32),
            pltpu.VMEM((_NBUF, _CHUNK, _D), jnp.float32),
            pltpu.SemaphoreType.DMA((_NBUF,)),
            pltpu.SemaphoreType.DMA((_NBUF,)),
        ],
    )(x, W1, b1_2d, W2, b2_2d)
